# baseline (speedup 1.0000x reference)
import functools

import jax
import jax.numpy as jnp
from jax import lax
from jax.experimental import pallas as pl
from jax.experimental.pallas import tpu as pltpu
from jax.experimental.pallas import tpu_sc as plsc

_RATIO = 0.25
_RADIUS = 0.2
_K = 32


def _fps_body(S, xyz_ref, o_ref, oi_ref):
    x = xyz_ref[0]
    y = xyz_ref[1]
    z = xyz_ref[2]
    B, N = x.shape
    iota = lax.broadcasted_iota(jnp.int32, (B, N), 1)
    iota_s = lax.broadcasted_iota(jnp.int32, (B, S), 1)
    boff = lax.broadcasted_iota(jnp.int32, (B, 1), 0) * N

    def body(i, st):
        dists, far, ax, ay, az, ai = st
        sel = iota == far
        cx = jnp.sum(jnp.where(sel, x, 0.0), axis=1, keepdims=True)
        cy = jnp.sum(jnp.where(sel, y, 0.0), axis=1, keepdims=True)
        cz = jnp.sum(jnp.where(sel, z, 0.0), axis=1, keepdims=True)
        out_sel = iota_s == i
        ax = jnp.where(out_sel, cx, ax)
        ay = jnp.where(out_sel, cy, ay)
        az = jnp.where(out_sel, cz, az)
        ai = jnp.where(out_sel, far + boff, ai)
        dx = x - cx
        dy = y - cy
        dz = z - cz
        d = (dx * dx + dy * dy) + dz * dz
        dists = jnp.minimum(dists, d)
        m = jnp.max(dists, axis=1, keepdims=True)
        far2 = jnp.min(jnp.where(dists == m, iota, N), axis=1, keepdims=True)
        return dists, far2, ax, ay, az, ai

    zs = jnp.zeros((B, S), jnp.float32)
    init = (jnp.full((B, N), 1e10, jnp.float32), jnp.zeros((B, 1), jnp.int32),
            zs, zs, zs, jnp.zeros((B, S), jnp.int32))
    _, _, ax, ay, az, ai = lax.fori_loop(0, S, body, init)
    o_ref[0] = ax
    o_ref[1] = ay
    o_ref[2] = az
    oi_ref[...] = ai


def _mlp_body(x_ref, w1_ref, b1_ref, w2_ref, b2_ref, w3_ref, b3_ref, o_ref):
    x = x_ref[...]
    h = jnp.maximum(jnp.dot(x, w1_ref[...], preferred_element_type=jnp.float32)
                    + b1_ref[...], 0.0)
    h = jnp.maximum(jnp.dot(h, w2_ref[...], preferred_element_type=jnp.float32)
                    + b2_ref[...], 0.0)
    h = jnp.maximum(jnp.dot(h, w3_ref[...], preferred_element_type=jnp.float32)
                    + b3_ref[...], 0.0)
    o_ref[...] = h


def _sc_group_body(N, S, B, xyzf, newxf, pfg, ft, out,
                   xv, yv, zv, cxs, cys, czs, pfv, slots, rows, outblk, sem):
    nw = 32
    sper = S * B // nw
    wid = lax.axis_index("s") * 2 + lax.axis_index("c")
    b = wid // 4
    s0 = pl.multiple_of(b * S + (wid % 4) * sper, sper)

    pltpu.sync_copy(xyzf.at[pl.ds(pl.multiple_of(b * N, N), N)], xv)
    pltpu.sync_copy(xyzf.at[pl.ds(pl.multiple_of((B + b) * N, N), N)], yv)
    pltpu.sync_copy(xyzf.at[pl.ds(pl.multiple_of((2 * B + b) * N, N), N)], zv)
    pltpu.sync_copy(newxf.at[pl.ds(s0, sper)], cxs)
    pltpu.sync_copy(newxf.at[pl.ds(B * S + s0, sper)], cys)
    pltpu.sync_copy(newxf.at[pl.ds(2 * B * S + s0, sper)], czs)
    pltpu.sync_copy(pfg.at[pl.ds(s0, sper)], pfv)

    lane = lax.iota(jnp.int32, 16)
    r2 = jnp.float32(_RADIUS ** 2)
    bn = b * N
    nch = N // 16

    def cent_body(i, carry):
        iv = jnp.full((16,), i, jnp.int32)
        cx = plsc.load_gather(cxs, [iv])
        cy = plsc.load_gather(cys, [iv])
        cz = plsc.load_gather(czs, [iv])
        pf16 = plsc.load_gather(pfv, [iv])
        slots[pl.ds(0, 16)] = pf16
        slots[pl.ds(16, 16)] = pf16

        def scan_cond(st):
            ch, cnt = st
            return (ch < nch) & (cnt < _K)

        def scan_body(st):
            ch, cnt = st
            off = pl.multiple_of(ch * 16, 16)
            px = xv[pl.ds(off, 16)]
            py = yv[pl.ds(off, 16)]
            pz = zv[pl.ds(off, 16)]
            dx = px - cx
            dy = py - cy
            dz = pz - cz
            d2 = (dx * dx + dy * dy) + dz * dz
            m = d2 <= r2
            within = plsc.cumsum(jnp.where(m, 1, 0).astype(jnp.int32))
            pos = within + (cnt - 1)
            act = m & (pos < _K)
            vals = lane + (ch * 16 + bn)
            plsc.store_scatter(slots, [pos], vals, mask=act)
            return ch + 1, cnt + jnp.max(within)

        lax.while_loop(scan_cond, scan_body, (jnp.int32(0), jnp.int32(0)))

        pltpu.async_copy(ft.at[slots], rows, sem).wait()
        for g in range(8):
            a = rows[0, pl.ds(g * 16, 16)]
            for r in range(1, _K):
                a = jnp.maximum(a, rows[r, pl.ds(g * 16, 16)])
            outblk[pl.ds(pl.multiple_of(i * 128 + g * 16, 16), 16)] = a
        return carry

    lax.fori_loop(0, sper, cent_body, jnp.int32(0))
    pltpu.sync_copy(outblk, out.at[pl.ds(s0 * 128, sper * 128)])


def kernel(xyz, features, W1, b1, W2, b2, W3, b3):
    B, N, _ = xyz.shape
    D = features.shape[-1]
    S = int(N * _RATIO)
    BN = B * N

    xyz_c = jnp.transpose(xyz, (2, 0, 1))
    new_xyz_c, fps_gidx = pl.pallas_call(
        functools.partial(_fps_body, S),
        out_shape=(jax.ShapeDtypeStruct((3, B, S), jnp.float32),
                   jax.ShapeDtypeStruct((B, S), jnp.int32)),
    )(xyz_c)

    feat2 = features.reshape(BN, D)
    rb = 4096
    ft = pl.pallas_call(
        _mlp_body,
        grid=(BN // rb,),
        in_specs=[
            pl.BlockSpec((rb, D), lambda i: (i, 0)),
            pl.BlockSpec((D, 64), lambda i: (0, 0)),
            pl.BlockSpec((1, 64), lambda i: (0, 0)),
            pl.BlockSpec((64, 64), lambda i: (0, 0)),
            pl.BlockSpec((1, 64), lambda i: (0, 0)),
            pl.BlockSpec((64, 128), lambda i: (0, 0)),
            pl.BlockSpec((1, 128), lambda i: (0, 0)),
        ],
        out_specs=pl.BlockSpec((rb, 128), lambda i: (i, 0)),
        out_shape=jax.ShapeDtypeStruct((BN, 128), jnp.float32),
    )(feat2, W1.T, b1.reshape(1, 64), W2.T, b2.reshape(1, 64),
      W3.T, b3.reshape(1, 128))

    new_xyz_out = jnp.transpose(new_xyz_c, (1, 0, 2))

    sper = S * B // 32
    sc_group = functools.partial(
        pl.kernel,
        out_type=jax.ShapeDtypeStruct((B * S * 128,), jnp.float32),
        mesh=plsc.VectorSubcoreMesh(core_axis_name="c", subcore_axis_name="s"),
        compiler_params=pltpu.CompilerParams(needs_layout_passes=False),
        scratch_types=[
            pltpu.VMEM((N,), jnp.float32),
            pltpu.VMEM((N,), jnp.float32),
            pltpu.VMEM((N,), jnp.float32),
            pltpu.VMEM((sper,), jnp.float32),
            pltpu.VMEM((sper,), jnp.float32),
            pltpu.VMEM((sper,), jnp.float32),
            pltpu.VMEM((sper,), jnp.int32),
            pltpu.VMEM((_K,), jnp.int32),
            pltpu.VMEM((_K, 128), jnp.float32),
            pltpu.VMEM((sper * 128,), jnp.float32),
            pltpu.SemaphoreType.DMA,
        ],
    )(functools.partial(_sc_group_body, N, S, B))
    flat = sc_group(xyz_c.reshape(3 * B * N), new_xyz_c.reshape(3 * B * S),
                    fps_gidx.reshape(B * S), ft)
    new_points = jnp.transpose(flat.reshape(B, S, 128), (0, 2, 1))

    return (new_xyz_out, new_points)


# speedup vs baseline: 1.0834x; 1.0834x over previous
import functools

import jax
import jax.numpy as jnp
from jax import lax
from jax.experimental import pallas as pl
from jax.experimental.pallas import tpu as pltpu
from jax.experimental.pallas import tpu_sc as plsc

_RATIO = 0.25
_RADIUS = 0.2
_K = 32


def _fps_body(S, xyz_ref, o_ref, oi_ref):
    x = xyz_ref[0]
    y = xyz_ref[1]
    z = xyz_ref[2]
    B, N = x.shape
    iota = lax.broadcasted_iota(jnp.int32, (B, N), 1)
    iota_s = lax.broadcasted_iota(jnp.int32, (B, S), 1)
    boff = lax.broadcasted_iota(jnp.int32, (B, 1), 0) * N

    def body(i, st):
        dists, far, ax, ay, az, ai = st
        sel = iota == far
        cx = jnp.sum(jnp.where(sel, x, 0.0), axis=1, keepdims=True)
        cy = jnp.sum(jnp.where(sel, y, 0.0), axis=1, keepdims=True)
        cz = jnp.sum(jnp.where(sel, z, 0.0), axis=1, keepdims=True)
        out_sel = iota_s == i
        ax = jnp.where(out_sel, cx, ax)
        ay = jnp.where(out_sel, cy, ay)
        az = jnp.where(out_sel, cz, az)
        ai = jnp.where(out_sel, far + boff, ai)
        dx = x - cx
        dy = y - cy
        dz = z - cz
        d = (dx * dx + dy * dy) + dz * dz
        dists = jnp.minimum(dists, d)
        m = jnp.max(dists, axis=1, keepdims=True)
        far2 = jnp.min(jnp.where(dists == m, iota, N), axis=1, keepdims=True)
        return dists, far2, ax, ay, az, ai

    zs = jnp.zeros((B, S), jnp.float32)
    init = (jnp.full((B, N), 1e10, jnp.float32), jnp.zeros((B, 1), jnp.int32),
            zs, zs, zs, jnp.zeros((B, S), jnp.int32))
    _, _, ax, ay, az, ai = lax.fori_loop(0, S, body, init)
    o_ref[0] = ax
    o_ref[1] = ay
    o_ref[2] = az
    oi_ref[...] = ai


def _mlp_body(x_ref, w1_ref, b1_ref, w2_ref, b2_ref, w3_ref, b3_ref, o_ref):
    x = x_ref[...]
    h = jnp.maximum(jnp.dot(x, w1_ref[...], preferred_element_type=jnp.float32)
                    + b1_ref[...], 0.0)
    h = jnp.maximum(jnp.dot(h, w2_ref[...], preferred_element_type=jnp.float32)
                    + b2_ref[...], 0.0)
    h = jnp.maximum(jnp.dot(h, w3_ref[...], preferred_element_type=jnp.float32)
                    + b3_ref[...], 0.0)
    o_ref[...] = h


def _sc_group_body(N, S, B, xyzf, newxf, pfg, ft, out,
                   xv, yv, zv, cxs, cys, czs, pfv, slots, rows, slots2, rows2,
                   outblk, sem, sem2):
    nw = 32
    sper = S * B // nw
    wid = lax.axis_index("s") * 2 + lax.axis_index("c")
    b = wid // 4
    s0 = pl.multiple_of(b * S + (wid % 4) * sper, sper)

    pltpu.sync_copy(xyzf.at[pl.ds(pl.multiple_of(b * N, N), N)], xv)
    pltpu.sync_copy(xyzf.at[pl.ds(pl.multiple_of((B + b) * N, N), N)], yv)
    pltpu.sync_copy(xyzf.at[pl.ds(pl.multiple_of((2 * B + b) * N, N), N)], zv)
    pltpu.sync_copy(newxf.at[pl.ds(s0, sper)], cxs)
    pltpu.sync_copy(newxf.at[pl.ds(B * S + s0, sper)], cys)
    pltpu.sync_copy(newxf.at[pl.ds(2 * B * S + s0, sper)], czs)
    pltpu.sync_copy(pfg.at[pl.ds(s0, sper)], pfv)

    lane = lax.iota(jnp.int32, 16)
    r2 = jnp.float32(_RADIUS ** 2)
    bn = b * N
    nch = N // 16

    def scan_into(i, sl):
        iv = jnp.full((16,), i, jnp.int32)
        cx = plsc.load_gather(cxs, [iv])
        cy = plsc.load_gather(cys, [iv])
        cz = plsc.load_gather(czs, [iv])
        pf16 = plsc.load_gather(pfv, [iv])
        sl[pl.ds(0, 16)] = pf16
        sl[pl.ds(16, 16)] = pf16

        def scan_cond(st):
            ch, cnt = st
            return (ch < nch) & (cnt < _K)

        def scan_body(st):
            ch, cnt = st
            off = pl.multiple_of(ch * 16, 16)
            px = xv[pl.ds(off, 16)]
            py = yv[pl.ds(off, 16)]
            pz = zv[pl.ds(off, 16)]
            dx = px - cx
            dy = py - cy
            dz = pz - cz
            d2 = (dx * dx + dy * dy) + dz * dz
            m = d2 <= r2
            within = plsc.cumsum(jnp.where(m, 1, 0).astype(jnp.int32))
            pos = within + (cnt - 1)
            act = m & (pos < _K)
            vals = lane + (ch * 16 + bn)
            plsc.store_scatter(sl, [pos], vals, mask=act)
            return ch + 1, cnt + jnp.max(within)

        lax.while_loop(scan_cond, scan_body, (jnp.int32(0), jnp.int32(0)))

    def max_out(i, rw):
        for g in range(8):
            a = rw[0, pl.ds(g * 16, 16)]
            for r in range(1, _K):
                a = jnp.maximum(a, rw[r, pl.ds(g * 16, 16)])
            outblk[pl.ds(pl.multiple_of(i * 128 + g * 16, 16), 16)] = a

    def pair_body(j, carry):
        i0 = pl.multiple_of(j * 2, 2)
        i1 = i0 + 1
        scan_into(i0, slots)
        da = pltpu.async_copy(ft.at[slots], rows, sem)
        scan_into(i1, slots2)
        db = pltpu.async_copy(ft.at[slots2], rows2, sem2)
        da.wait()
        max_out(i0, rows)
        db.wait()
        max_out(i1, rows2)
        return carry

    lax.fori_loop(0, sper // 2, pair_body, jnp.int32(0))
    pltpu.sync_copy(outblk, out.at[pl.ds(s0 * 128, sper * 128)])


def kernel(xyz, features, W1, b1, W2, b2, W3, b3):
    B, N, _ = xyz.shape
    D = features.shape[-1]
    S = int(N * _RATIO)
    BN = B * N

    xyz_c = jnp.transpose(xyz, (2, 0, 1))
    new_xyz_c, fps_gidx = pl.pallas_call(
        functools.partial(_fps_body, S),
        out_shape=(jax.ShapeDtypeStruct((3, B, S), jnp.float32),
                   jax.ShapeDtypeStruct((B, S), jnp.int32)),
    )(xyz_c)

    feat2 = features.reshape(BN, D)
    rb = 4096
    ft = pl.pallas_call(
        _mlp_body,
        grid=(BN // rb,),
        in_specs=[
            pl.BlockSpec((rb, D), lambda i: (i, 0)),
            pl.BlockSpec((D, 64), lambda i: (0, 0)),
            pl.BlockSpec((1, 64), lambda i: (0, 0)),
            pl.BlockSpec((64, 64), lambda i: (0, 0)),
            pl.BlockSpec((1, 64), lambda i: (0, 0)),
            pl.BlockSpec((64, 128), lambda i: (0, 0)),
            pl.BlockSpec((1, 128), lambda i: (0, 0)),
        ],
        out_specs=pl.BlockSpec((rb, 128), lambda i: (i, 0)),
        out_shape=jax.ShapeDtypeStruct((BN, 128), jnp.float32),
    )(feat2, W1.T, b1.reshape(1, 64), W2.T, b2.reshape(1, 64),
      W3.T, b3.reshape(1, 128))

    new_xyz_out = jnp.transpose(new_xyz_c, (1, 0, 2))

    sper = S * B // 32
    sc_group = functools.partial(
        pl.kernel,
        out_type=jax.ShapeDtypeStruct((B * S * 128,), jnp.float32),
        mesh=plsc.VectorSubcoreMesh(core_axis_name="c", subcore_axis_name="s"),
        compiler_params=pltpu.CompilerParams(needs_layout_passes=False),
        scratch_types=[
            pltpu.VMEM((N,), jnp.float32),
            pltpu.VMEM((N,), jnp.float32),
            pltpu.VMEM((N,), jnp.float32),
            pltpu.VMEM((sper,), jnp.float32),
            pltpu.VMEM((sper,), jnp.float32),
            pltpu.VMEM((sper,), jnp.float32),
            pltpu.VMEM((sper,), jnp.int32),
            pltpu.VMEM((_K,), jnp.int32),
            pltpu.VMEM((_K, 128), jnp.float32),
            pltpu.VMEM((_K,), jnp.int32),
            pltpu.VMEM((_K, 128), jnp.float32),
            pltpu.VMEM((sper * 128,), jnp.float32),
            pltpu.SemaphoreType.DMA,
            pltpu.SemaphoreType.DMA,
        ],
    )(functools.partial(_sc_group_body, N, S, B))
    flat = sc_group(xyz_c.reshape(3 * B * N), new_xyz_c.reshape(3 * B * S),
                    fps_gidx.reshape(B * S), ft)
    new_points = jnp.transpose(flat.reshape(B, S, 128), (0, 2, 1))

    return (new_xyz_out, new_points)
